# Initial kernel scaffold; baseline (speedup 1.0000x reference)
import jax
import jax.numpy as jnp
from jax.experimental import pallas as pl


def kernel(x, edge_index, init_W0, init_b0, init_W1, init_b1, init_W2, init_b2, node_W0, node_b0, node_W1, node_b1, node_W2, node_b2):
    raise NotImplementedError("write your pallas kernel here")


# speedup vs baseline: 3.3135x; 3.3135x over previous
import functools

import jax
import jax.numpy as jnp
from jax import lax
from jax.experimental import pallas as pl
from jax.experimental.pallas import tpu as pltpu
from jax.experimental.pallas import tpu_sc as plsc

D = 128
NC = 2
NS = 16
GROUP = 128
NITER = 3


def _make_sc_scatter(n_nodes, gpw):
    n_pad = ((n_nodes + 1 + NS * 8 - 1) // (NS * 8)) * (NS * 8)
    rpt = n_pad // NS
    gph = gpw // 2
    mesh = plsc.VectorSubcoreMesh(core_axis_name="c", subcore_axis_name="s")

    @functools.partial(
        pl.kernel,
        mesh=mesh,
        out_type=jax.ShapeDtypeStruct((NC, n_pad, D), jnp.float32),
        scratch_types=[
            pltpu.VMEM((gph, GROUP), jnp.int32),
            pltpu.VMEM((gph, GROUP), jnp.int32),
            pltpu.VMEM((GROUP, D), jnp.float32),
            pltpu.VMEM((GROUP, D), jnp.float32),
            pltpu.VMEM_SHARED((n_pad, D), jnp.float32),
            pltpu.SemaphoreType.DMA,
            pltpu.SemaphoreType.DMA,
        ],
    )
    def sc_scatter(h_hbm, src_hbm, dst_hbm, out_hbm,
                   src_v, dst_v, rows0, rows1, agg_sp, sem0, sem1):
        cid = lax.axis_index("c")
        sid = lax.axis_index("s")
        wid = cid * NS + sid

        def zrow(i, carry):
            for j in range(D // 16):
                rows0[i, pl.ds(j * 16, 16)] = jnp.zeros((16,), jnp.float32)
            return carry
        lax.fori_loop(0, GROUP, zrow, 0)
        for k in range(0, rpt, GROUP):
            c = min(GROUP, rpt - k)
            pltpu.sync_copy(rows0.at[pl.ds(0, c)],
                            agg_sp.at[pl.ds(sid * rpt + k, c)])
        plsc.subcore_barrier()

        def body(p, carry):
            g0 = 2 * p
            g1 = g0 + 1
            pltpu.async_copy(h_hbm.at[src_v.at[g1]], rows1, sem1)
            pltpu.make_async_copy(h_hbm.at[src_v.at[g0]], rows0, sem0).wait()
            pltpu.sync_copy(rows0, agg_sp.at[dst_v.at[g0]], add=True)

            @pl.when(g0 + 2 < gph)
            def _():
                pltpu.async_copy(h_hbm.at[src_v.at[g0 + 2]], rows0, sem0)

            pltpu.make_async_copy(h_hbm.at[src_v.at[g1]], rows1, sem1).wait()
            pltpu.sync_copy(rows1, agg_sp.at[dst_v.at[g1]], add=True)
            return carry

        for half in range(2):
            pltpu.sync_copy(src_hbm.at[wid, half], src_v)
            pltpu.sync_copy(dst_hbm.at[wid, half], dst_v)
            pltpu.async_copy(h_hbm.at[src_v.at[0]], rows0, sem0)
            lax.fori_loop(0, gph // 2, body, 0)

        plsc.subcore_barrier()
        pltpu.sync_copy(agg_sp.at[pl.ds(sid * rpt, rpt)],
                        out_hbm.at[cid, pl.ds(sid * rpt, rpt)])

    return sc_scatter, n_pad


def _init_mlp_body(x_ref, w0, b0, w1, b1, w2, b2, h_ref, gs_ref):
    t = jnp.maximum(jnp.dot(x_ref[...], w0[...],
                            preferred_element_type=jnp.float32) + b0[...], 0.0)
    t = jnp.maximum(jnp.dot(t, w1[...],
                            preferred_element_type=jnp.float32) + b1[...], 0.0)
    h = jnp.dot(t, w2[...], preferred_element_type=jnp.float32) + b2[...]
    h_ref[...] = h

    @pl.when(pl.program_id(0) == 0)
    def _():
        gs_ref[...] = jnp.zeros_like(gs_ref)
    s = jnp.sum(h, axis=0, keepdims=True)
    gs_ref[...] += jnp.broadcast_to(s, gs_ref.shape)


def _node_mlp_body(parts_ref, h_ref, gsum_ref, w0, b0, w1, b1, w2, b2,
                   hout_ref, gs_ref):
    gs = gsum_ref[0:1, :]
    g = gs / (jnp.sqrt(jnp.sum(gs * gs)) + 1e-8)
    c = jnp.dot(g, w0[2 * D:3 * D, :],
                preferred_element_type=jnp.float32) + b0[...]
    agg = parts_ref[0] + parts_ref[1]
    t = jnp.maximum(
        jnp.dot(agg, w0[0:D, :], preferred_element_type=jnp.float32)
        + jnp.dot(h_ref[...], w0[D:2 * D, :], preferred_element_type=jnp.float32)
        + c, 0.0)
    t = jnp.maximum(jnp.dot(t, w1[...],
                            preferred_element_type=jnp.float32) + b1[...], 0.0)
    o = jnp.dot(t, w2[...], preferred_element_type=jnp.float32) + b2[...]
    o = o / (jnp.sqrt(jnp.sum(o * o, axis=1, keepdims=True)) + 1e-8)
    hout_ref[...] = o

    @pl.when(pl.program_id(0) == 0)
    def _():
        gs_ref[...] = jnp.zeros_like(gs_ref)
    s = jnp.sum(o, axis=0, keepdims=True)
    gs_ref[...] += jnp.broadcast_to(s, gs_ref.shape)


def _full(shape):
    return pl.BlockSpec(shape, lambda i: (0,) * len(shape))


def _tc_init(x, w0, b0, w1, b1, w2, b2, bn):
    n = x.shape[0]
    grid = (n // bn,)
    return pl.pallas_call(
        _init_mlp_body,
        grid=grid,
        in_specs=[
            pl.BlockSpec((bn, D), lambda i: (i, 0)),
            _full((D, D)), _full((1, D)),
            _full((D, D)), _full((1, D)),
            _full((D, D)), _full((1, D)),
        ],
        out_specs=[
            pl.BlockSpec((bn, D), lambda i: (i, 0)),
            pl.BlockSpec((8, D), lambda i: (0, 0)),
        ],
        out_shape=[
            jax.ShapeDtypeStruct((n, D), jnp.float32),
            jax.ShapeDtypeStruct((8, D), jnp.float32),
        ],
    )(x, w0, b0, w1, b1, w2, b2)


def _tc_node(parts, h, gsum, w0, b0, w1, b1, w2, b2, bn):
    n = h.shape[0]
    grid = (n // bn,)
    return pl.pallas_call(
        _node_mlp_body,
        grid=grid,
        in_specs=[
            pl.BlockSpec((NC, bn, D), lambda i: (0, i, 0)),
            pl.BlockSpec((bn, D), lambda i: (i, 0)),
            _full((8, D)),
            _full((3 * D, D)), _full((1, D)),
            _full((D, D)), _full((1, D)),
            _full((D, D)), _full((1, D)),
        ],
        out_specs=[
            pl.BlockSpec((bn, D), lambda i: (i, 0)),
            pl.BlockSpec((8, D), lambda i: (0, 0)),
        ],
        out_shape=[
            jax.ShapeDtypeStruct((n, D), jnp.float32),
            jax.ShapeDtypeStruct((8, D), jnp.float32),
        ],
    )(parts, h, gsum, w0, b0, w1, b1, w2, b2)


def kernel(x, edge_index, init_W0, init_b0, init_W1, init_b1, init_W2,
           init_b2, node_W0, node_b0, node_W1, node_b1, node_W2, node_b2):
    n = x.shape[0]
    e = edge_index.shape[1]
    bn = 400

    gpw = -(-e // (NC * NS * GROUP))
    gpw += (-gpw) % 4
    e_pad = NC * NS * gpw * GROUP
    src = edge_index[0]
    dst = edge_index[1]
    if e_pad > e:
        pad = e_pad - e
        src = jnp.concatenate([src, jnp.zeros((pad,), jnp.int32)])
        dst = jnp.concatenate([dst, jnp.full((pad,), n, jnp.int32)])
    src3 = src.reshape(NC * NS, 2, gpw // 2, GROUP)
    dst3 = dst.reshape(NC * NS, 2, gpw // 2, GROUP)
    src3, dst3 = lax.optimization_barrier((src3, dst3))

    sc_scatter, _ = _make_sc_scatter(n, gpw)

    b0i = init_b0.reshape(1, D)
    b1i = init_b1.reshape(1, D)
    b2i = init_b2.reshape(1, D)

    h, gsum = _tc_init(x, init_W0, b0i, init_W1, b1i, init_W2, b2i, bn)
    for i in range(NITER):
        parts = sc_scatter(h, src3, dst3)
        h, gsum = _tc_node(parts, h, gsum,
                           node_W0[i], node_b0[i].reshape(1, D),
                           node_W1[i], node_b1[i].reshape(1, D),
                           node_W2[i], node_b2[i].reshape(1, D), bn)
    return h


# speedup vs baseline: 11.0863x; 3.3458x over previous
import functools

import jax
import jax.numpy as jnp
from jax import lax
from jax.experimental import pallas as pl
from jax.experimental.pallas import tpu as pltpu
from jax.experimental.pallas import tpu_sc as plsc

D = 128
NC = 2
NS = 16
GROUP = 128
NITER = 3


def _make_sc_scatter(n_nodes, gpw):
    n_pad = ((n_nodes + 1 + NS * 8 - 1) // (NS * 8)) * (NS * 8)
    rpt = n_pad // NS
    gph = gpw // 2
    mesh = plsc.VectorSubcoreMesh(core_axis_name="c", subcore_axis_name="s")

    @functools.partial(
        pl.kernel,
        mesh=mesh,
        out_type=jax.ShapeDtypeStruct((NC, n_pad, D), jnp.float32),
        scratch_types=[
            pltpu.VMEM((gph, GROUP), jnp.int32),
            pltpu.VMEM((gph, GROUP), jnp.int32),
            pltpu.VMEM((GROUP, D), jnp.float32),
            pltpu.VMEM((GROUP, D), jnp.float32),
            pltpu.VMEM_SHARED((n_pad, D), jnp.float32),
            pltpu.SemaphoreType.DMA,
            pltpu.SemaphoreType.DMA,
        ],
    )
    def sc_scatter(h_hbm, src_hbm, dst_hbm, out_hbm,
                   src_v, dst_v, rows0, rows1, agg_sp, sem0, sem1):
        cid = lax.axis_index("c")
        sid = lax.axis_index("s")
        wid = cid * NS + sid

        def zrow(i, carry):
            for j in range(D // 16):
                rows0[i, pl.ds(j * 16, 16)] = jnp.zeros((16,), jnp.float32)
            return carry
        lax.fori_loop(0, GROUP, zrow, 0)
        for k in range(0, rpt, GROUP):
            c = min(GROUP, rpt - k)
            pltpu.sync_copy(rows0.at[pl.ds(0, c)],
                            agg_sp.at[pl.ds(sid * rpt + k, c)])
        plsc.subcore_barrier()

        def body(p, carry):
            g0 = 2 * p
            g1 = g0 + 1
            pltpu.async_copy(h_hbm.at[src_v.at[g1]], rows1, sem1)
            pltpu.make_async_copy(h_hbm.at[src_v.at[g0]], rows0, sem0).wait()
            pltpu.sync_copy(rows0, agg_sp.at[dst_v.at[g0]], add=True)

            @pl.when(g0 + 2 < gph)
            def _():
                pltpu.async_copy(h_hbm.at[src_v.at[g0 + 2]], rows0, sem0)

            pltpu.make_async_copy(h_hbm.at[src_v.at[g1]], rows1, sem1).wait()
            pltpu.sync_copy(rows1, agg_sp.at[dst_v.at[g1]], add=True)
            return carry

        for half in range(2):
            pltpu.sync_copy(src_hbm.at[wid, half], src_v)
            pltpu.sync_copy(dst_hbm.at[wid, half], dst_v)
            pltpu.async_copy(h_hbm.at[src_v.at[0]], rows0, sem0)
            lax.fori_loop(0, gph // 2, body, 0)

        plsc.subcore_barrier()
        pltpu.sync_copy(agg_sp.at[pl.ds(sid * rpt, rpt)],
                        out_hbm.at[cid, pl.ds(sid * rpt, rpt)])

    return sc_scatter, n_pad


def _init_mlp_body(x_ref, w0, b0, w1, b1, w2, b2, h_ref, gs_ref):
    t = jnp.maximum(jnp.dot(x_ref[...], w0[...],
                            preferred_element_type=jnp.float32) + b0[...], 0.0)
    t = jnp.maximum(jnp.dot(t, w1[...],
                            preferred_element_type=jnp.float32) + b1[...], 0.0)
    h = jnp.dot(t, w2[...], preferred_element_type=jnp.float32) + b2[...]
    h_ref[...] = h

    @pl.when(pl.program_id(0) == 0)
    def _():
        gs_ref[...] = jnp.zeros_like(gs_ref)
    s = jnp.sum(h, axis=0, keepdims=True)
    gs_ref[...] += jnp.broadcast_to(s, gs_ref.shape)


def _node_mlp_body(parts_ref, h_ref, gsum_ref, w0, b0, w1, b1, w2, b2,
                   hout_ref, gs_ref):
    gs = gsum_ref[0:1, :]
    g = gs / (jnp.sqrt(jnp.sum(gs * gs)) + 1e-8)
    c = jnp.dot(g, w0[2 * D:3 * D, :],
                preferred_element_type=jnp.float32) + b0[...]
    agg = parts_ref[0] + parts_ref[1]
    t = jnp.maximum(
        jnp.dot(agg, w0[0:D, :], preferred_element_type=jnp.float32)
        + jnp.dot(h_ref[...], w0[D:2 * D, :], preferred_element_type=jnp.float32)
        + c, 0.0)
    t = jnp.maximum(jnp.dot(t, w1[...],
                            preferred_element_type=jnp.float32) + b1[...], 0.0)
    o = jnp.dot(t, w2[...], preferred_element_type=jnp.float32) + b2[...]
    o = o / (jnp.sqrt(jnp.sum(o * o, axis=1, keepdims=True)) + 1e-8)
    hout_ref[...] = o

    @pl.when(pl.program_id(0) == 0)
    def _():
        gs_ref[...] = jnp.zeros_like(gs_ref)
    s = jnp.sum(o, axis=0, keepdims=True)
    gs_ref[...] += jnp.broadcast_to(s, gs_ref.shape)


def _full(shape):
    return pl.BlockSpec(shape, lambda i: (0,) * len(shape))


def _tc_init(x, w0, b0, w1, b1, w2, b2, bn):
    n = x.shape[0]
    grid = (n // bn,)
    return pl.pallas_call(
        _init_mlp_body,
        grid=grid,
        in_specs=[
            pl.BlockSpec((bn, D), lambda i: (i, 0)),
            _full((D, D)), _full((1, D)),
            _full((D, D)), _full((1, D)),
            _full((D, D)), _full((1, D)),
        ],
        out_specs=[
            pl.BlockSpec((bn, D), lambda i: (i, 0)),
            pl.BlockSpec((8, D), lambda i: (0, 0)),
        ],
        out_shape=[
            jax.ShapeDtypeStruct((n, D), jnp.float32),
            jax.ShapeDtypeStruct((8, D), jnp.float32),
        ],
    )(x, w0, b0, w1, b1, w2, b2)


def _tc_node(parts, h, gsum, w0, b0, w1, b1, w2, b2, bn):
    n = h.shape[0]
    grid = (n // bn,)
    return pl.pallas_call(
        _node_mlp_body,
        grid=grid,
        in_specs=[
            pl.BlockSpec((NC, bn, D), lambda i: (0, i, 0)),
            pl.BlockSpec((bn, D), lambda i: (i, 0)),
            _full((8, D)),
            _full((3 * D, D)), _full((1, D)),
            _full((D, D)), _full((1, D)),
            _full((D, D)), _full((1, D)),
        ],
        out_specs=[
            pl.BlockSpec((bn, D), lambda i: (i, 0)),
            pl.BlockSpec((8, D), lambda i: (0, 0)),
        ],
        out_shape=[
            jax.ShapeDtypeStruct((n, D), jnp.float32),
            jax.ShapeDtypeStruct((8, D), jnp.float32),
        ],
    )(parts, h, gsum, w0, b0, w1, b1, w2, b2)


def kernel(x, edge_index, init_W0, init_b0, init_W1, init_b1, init_W2,
           init_b2, node_W0, node_b0, node_W1, node_b1, node_W2, node_b2):
    n = x.shape[0]
    e = edge_index.shape[1]
    bn = 400

    gpw = -(-e // (NC * NS * GROUP))
    gpw += (-gpw) % 4
    e_pad = NC * NS * gpw * GROUP
    src = edge_index[0]
    dst = edge_index[1]
    if e_pad > e:
        pad = e_pad - e
        n_pad = ((n + 1 + NS * 8 - 1) // (NS * 8)) * (NS * 8)
        ar = jnp.arange(pad, dtype=jnp.int32)
        src = jnp.concatenate([src, ar % jnp.int32(n)])
        dst = jnp.concatenate([dst, jnp.int32(n) + ar % jnp.int32(n_pad - n)])
    src3 = src.reshape(NC * NS, 2, gpw // 2, GROUP)
    dst3 = dst.reshape(NC * NS, 2, gpw // 2, GROUP)
    src3, dst3 = lax.optimization_barrier((src3, dst3))

    sc_scatter, _ = _make_sc_scatter(n, gpw)

    b0i = init_b0.reshape(1, D)
    b1i = init_b1.reshape(1, D)
    b2i = init_b2.reshape(1, D)

    h, gsum = _tc_init(x, init_W0, b0i, init_W1, b1i, init_W2, b2i, bn)
    for i in range(NITER):
        parts = sc_scatter(h, src3, dst3)
        h, gsum = _tc_node(parts, h, gsum,
                           node_W0[i], node_b0[i].reshape(1, D),
                           node_W1[i], node_b1[i].reshape(1, D),
                           node_W2[i], node_b2[i].reshape(1, D), bn)
    return h
